# Initial kernel scaffold; baseline (speedup 1.0000x reference)
import jax
import jax.numpy as jnp
from jax.experimental import pallas as pl


def kernel(x, edge_index, W_s, W_n):
    raise NotImplementedError("write your pallas kernel here")


# speedup vs baseline: 5.4847x; 5.4847x over previous
import functools

import jax
import jax.numpy as jnp
from jax import lax
from jax.experimental import pallas as pl
from jax.experimental.pallas import tpu as pltpu
from jax.experimental.pallas import tpu_sc as plsc

N_NODES = 10000
N_EDGES = 320000
D_FEAT = 128

NC = 2
NS = 16
NW = NC * NS
EDGES_PER_TILE = N_EDGES // NW
CHUNK = 80
NCHUNKS = EDGES_PER_TILE // CHUNK
ROWS_PER_TILE = 640
N_PAD = NS * ROWS_PER_TILE


def _sc_segment_sum(x, src, dst, zeros):
    mesh = plsc.VectorSubcoreMesh(core_axis_name="c", subcore_axis_name="s")

    @functools.partial(
        pl.kernel,
        mesh=mesh,
        out_type=jax.ShapeDtypeStruct((NC, N_PAD, D_FEAT), jnp.float32),
        scratch_types=[
            pltpu.VMEM_SHARED((N_PAD, D_FEAT), jnp.float32),
            pltpu.VMEM((CHUNK,), jnp.int32),
            pltpu.VMEM((CHUNK,), jnp.int32),
            pltpu.VMEM((CHUNK, D_FEAT), jnp.float32),
            pltpu.SemaphoreType.DMA,
        ],
    )
    def k(x_hbm, src_hbm, dst_hbm, zeros_hbm, out_hbm, acc_sh, sidx, didx,
          rows, sem):
        cid = lax.axis_index("c")
        sid = lax.axis_index("s")
        wid = cid * NS + sid
        slab = pl.ds(sid * ROWS_PER_TILE, ROWS_PER_TILE)
        pltpu.sync_copy(zeros_hbm.at[slab], acc_sh.at[slab])
        plsc.subcore_barrier()

        base = wid * EDGES_PER_TILE

        def body(i, carry):
            off = base + i * CHUNK
            pltpu.sync_copy(src_hbm.at[pl.ds(off, CHUNK)], sidx)
            pltpu.sync_copy(dst_hbm.at[pl.ds(off, CHUNK)], didx)
            pltpu.async_copy(x_hbm.at[sidx], rows, sem).wait()
            pltpu.sync_copy(rows, acc_sh.at[didx], add=True)
            return carry

        lax.fori_loop(0, NCHUNKS, body, 0)
        plsc.subcore_barrier()
        pltpu.sync_copy(acc_sh.at[slab], out_hbm.at[cid, slab])

    return k(x, src, dst, zeros)


BLK_ROWS = 1000


def _tc_combine(x, partials, Wst, Wnt):
    def body(x_ref, p_ref, wst_ref, wnt_ref, o_ref):
        agg = p_ref[0] + p_ref[1]
        z = (jnp.dot(x_ref[...], wst_ref[...],
                     preferred_element_type=jnp.float32)
             + jnp.dot(agg, wnt_ref[...],
                       preferred_element_type=jnp.float32))
        o_ref[...] = 0.5 * z * (1.0 + lax.erf(z * 0.7071067811865476))

    grid = (N_NODES // BLK_ROWS,)
    return pl.pallas_call(
        body,
        grid=grid,
        in_specs=[
            pl.BlockSpec((BLK_ROWS, D_FEAT), lambda i: (i, 0)),
            pl.BlockSpec((NC, BLK_ROWS, D_FEAT), lambda i: (0, i, 0)),
            pl.BlockSpec((D_FEAT, D_FEAT), lambda i: (0, 0)),
            pl.BlockSpec((D_FEAT, D_FEAT), lambda i: (0, 0)),
        ],
        out_specs=pl.BlockSpec((BLK_ROWS, D_FEAT), lambda i: (i, 0)),
        out_shape=jax.ShapeDtypeStruct((N_NODES, D_FEAT), jnp.float32),
    )(x, partials, Wst, Wnt)


def kernel(x, edge_index, W_s, W_n):
    src = edge_index[0].astype(jnp.int32)
    dst = edge_index[1].astype(jnp.int32)
    zeros = jnp.zeros((N_PAD, D_FEAT), jnp.float32)
    partials = _sc_segment_sum(x, src, dst, zeros)
    return _tc_combine(x, partials, W_s.T, W_n.T)


# speedup vs baseline: 11.9086x; 2.1712x over previous
import functools

import jax
import jax.numpy as jnp
from jax import lax
from jax.experimental import pallas as pl
from jax.experimental.pallas import tpu as pltpu
from jax.experimental.pallas import tpu_sc as plsc

N_NODES = 10000
N_EDGES = 320000
D_FEAT = 128

NC = 2
NS = 16
NW = NC * NS
EDGES_PER_TILE = N_EDGES // NW
CHUNK = 80
NCHUNKS = EDGES_PER_TILE // CHUNK
ROWS_PER_TILE = 640
N_PAD = NS * ROWS_PER_TILE


def _sc_segment_sum(x, src, dst, zeros):
    mesh = plsc.VectorSubcoreMesh(core_axis_name="c", subcore_axis_name="s")

    @functools.partial(
        pl.kernel,
        mesh=mesh,
        out_type=jax.ShapeDtypeStruct((NC, N_PAD, D_FEAT), jnp.float32),
        scratch_types=[
            pltpu.VMEM_SHARED((N_PAD, D_FEAT), jnp.float32),
            pltpu.VMEM((EDGES_PER_TILE,), jnp.int32),
            pltpu.VMEM((CHUNK,), jnp.int32),
            pltpu.VMEM((CHUNK,), jnp.int32),
            pltpu.VMEM((CHUNK, D_FEAT), jnp.float32),
            pltpu.VMEM((CHUNK, D_FEAT), jnp.float32),
            pltpu.SemaphoreType.DMA,
            pltpu.SemaphoreType.DMA,
            pltpu.SemaphoreType.DMA,
            pltpu.SemaphoreType.DMA,
        ],
    )
    def k(x_hbm, src_hbm, dst_hbm, zeros_hbm, out_hbm, acc_sh, sidx, didx0,
          didx1, rows0, rows1, sem0, sem1, semd0, semd1):
        cid = lax.axis_index("c")
        sid = lax.axis_index("s")
        wid = cid * NS + sid
        slab = pl.ds(sid * ROWS_PER_TILE, ROWS_PER_TILE)
        ebase = wid * EDGES_PER_TILE
        pltpu.sync_copy(src_hbm.at[pl.ds(ebase, EDGES_PER_TILE)], sidx)
        pltpu.sync_copy(zeros_hbm.at[slab], acc_sh.at[slab])
        plsc.subcore_barrier()

        def sidx_c(i):
            return sidx.at[pl.ds(i * CHUNK, CHUNK)]

        def didx_c(i):
            return dst_hbm.at[pl.ds(ebase + i * CHUNK, CHUNK)]

        pltpu.async_copy(didx_c(0), didx0, semd0)
        pltpu.async_copy(x_hbm.at[sidx_c(0)], rows0, sem0)

        def body(j, carry):
            i = 2 * j
            pltpu.async_copy(didx_c(i + 1), didx1, semd1)
            pltpu.async_copy(x_hbm.at[sidx_c(i + 1)], rows1, sem1)
            pltpu.make_async_copy(x_hbm.at[sidx_c(i)], rows0, sem0).wait()
            pltpu.make_async_copy(didx_c(i), didx0, semd0).wait()
            pltpu.sync_copy(rows0, acc_sh.at[didx0], add=True)
            pltpu.async_copy(didx_c(i + 2), didx0, semd0)
            pltpu.async_copy(x_hbm.at[sidx_c(i + 2)], rows0, sem0)
            pltpu.make_async_copy(x_hbm.at[sidx_c(i + 1)], rows1, sem1).wait()
            pltpu.make_async_copy(didx_c(i + 1), didx1, semd1).wait()
            pltpu.sync_copy(rows1, acc_sh.at[didx1], add=True)
            return carry

        lax.fori_loop(0, (NCHUNKS - 1) // 2, body, 0)
        pltpu.make_async_copy(x_hbm.at[sidx_c(NCHUNKS - 1)], rows0,
                              sem0).wait()
        pltpu.make_async_copy(didx_c(NCHUNKS - 1), didx0, semd0).wait()
        pltpu.sync_copy(rows0, acc_sh.at[didx0], add=True)
        plsc.subcore_barrier()
        pltpu.sync_copy(acc_sh.at[slab], out_hbm.at[cid, slab])

    return k(x, src, dst, zeros)


BLK_ROWS = 1000


def _tc_combine(x, partials, Wst, Wnt):
    def body(x_ref, p_ref, wst_ref, wnt_ref, o_ref):
        agg = p_ref[0] + p_ref[1]
        z = (jnp.dot(x_ref[...], wst_ref[...],
                     preferred_element_type=jnp.float32)
             + jnp.dot(agg, wnt_ref[...],
                       preferred_element_type=jnp.float32))
        o_ref[...] = 0.5 * z * (1.0 + lax.erf(z * 0.7071067811865476))

    grid = (N_NODES // BLK_ROWS,)
    return pl.pallas_call(
        body,
        grid=grid,
        in_specs=[
            pl.BlockSpec((BLK_ROWS, D_FEAT), lambda i: (i, 0)),
            pl.BlockSpec((NC, BLK_ROWS, D_FEAT), lambda i: (0, i, 0)),
            pl.BlockSpec((D_FEAT, D_FEAT), lambda i: (0, 0)),
            pl.BlockSpec((D_FEAT, D_FEAT), lambda i: (0, 0)),
        ],
        out_specs=pl.BlockSpec((BLK_ROWS, D_FEAT), lambda i: (i, 0)),
        out_shape=jax.ShapeDtypeStruct((N_NODES, D_FEAT), jnp.float32),
    )(x, partials, Wst, Wnt)


def kernel(x, edge_index, W_s, W_n):
    src = edge_index[0].astype(jnp.int32)
    dst = edge_index[1].astype(jnp.int32)
    zeros = jnp.zeros((N_PAD, D_FEAT), jnp.float32)
    partials = _sc_segment_sum(x, src, dst, zeros)
    return _tc_combine(x, partials, W_s.T, W_n.T)
